# baseline (speedup 1.0000x reference)
import functools

import jax
import jax.numpy as jnp
from jax import lax
from jax.experimental import pallas as pl
from jax.experimental.pallas import tpu as pltpu
from jax.experimental.pallas import tpu_sc as plsc

_B, _S, _D = 1024, 50, 1000
_NR = _S * _D
_RPW = 1568
_CH = 48
_NCH = 32
_FIN = _RPW - _NCH * _CH


def _onehot_sc(xt_hbm, out_hbm, xbuf, stage, sem0, sem1):
    cid = lax.axis_index("c")
    sid = lax.axis_index("s")
    wid = sid * 2 + cid
    base = pl.multiple_of((wid * (_NR // 8) // 32) * 8, 8)

    s_first = base // _D
    s_last = (base + _RPW - 1) // _D
    a0 = pl.multiple_of((s_first // 8) * 8, 8)
    a1 = pl.multiple_of((s_last // 8) * 8, 8)
    pltpu.sync_copy(xt_hbm.at[pl.ds(a0, 8)], xbuf.at[pl.ds(0, 8)])
    pltpu.sync_copy(xt_hbm.at[pl.ds(a1, 8)], xbuf.at[pl.ds(8, 8)])

    sems = (sem0, sem1)

    def _fill(k, rowbase, nrows):
        def _row(rr, _):
            r = rowbase + rr
            s = r // _D
            d = r - s * _D
            xrow = jnp.minimum(s - a0, s - a1 + 8)
            for g in range(_B // 16):
                xg = xbuf[xrow, pl.ds(g * 16, 16)]
                stage[k, rr, pl.ds(g * 16, 16)] = jnp.where(xg == d, 1.0, 0.0).astype(
                    jnp.float32
                )
            return 0

        lax.fori_loop(0, nrows, _row, 0)

    for k in (0, 1):
        rb = pl.multiple_of(base + k * _CH, 8)
        _fill(k, rb, _CH)
        pltpu.async_copy(stage.at[k], out_hbm.at[pl.ds(rb, _CH)], sems[k])

    def _pair(cp, _):
        for k in (0, 1):
            rb = pl.multiple_of(base + (2 * cp + k) * _CH, 8)
            pltpu.make_async_copy(
                stage.at[k], out_hbm.at[pl.ds(base, _CH)], sems[k]
            ).wait()
            _fill(k, rb, _CH)
            pltpu.async_copy(stage.at[k], out_hbm.at[pl.ds(rb, _CH)], sems[k])
        return 0

    lax.fori_loop(1, _NCH // 2, _pair, 0)

    rb = pl.multiple_of(base + _NCH * _CH, 8)
    pltpu.make_async_copy(stage.at[0], out_hbm.at[pl.ds(base, _CH)], sems[0]).wait()
    _fill(0, rb, _FIN)
    pltpu.async_copy(
        stage.at[0, pl.ds(0, _FIN)], out_hbm.at[pl.ds(rb, _FIN)], sems[0]
    )
    pltpu.make_async_copy(
        stage.at[0, pl.ds(0, _FIN)], out_hbm.at[pl.ds(base, _FIN)], sems[0]
    ).wait()
    pltpu.make_async_copy(stage.at[1], out_hbm.at[pl.ds(base, _CH)], sems[1]).wait()


def kernel(x):
    xt = jnp.pad(x.astype(jnp.int32).T, ((0, 6), (0, 0)))
    mesh = plsc.VectorSubcoreMesh(
        core_axis_name="c", subcore_axis_name="s", num_cores=2
    )
    fn = functools.partial(
        pl.kernel,
        mesh=mesh,
        out_type=jax.ShapeDtypeStruct((_NR, _B), jnp.float32),
        scratch_types=[
            pltpu.VMEM((16, _B), jnp.int32),
            pltpu.VMEM((2, _CH, _B), jnp.float32),
            pltpu.SemaphoreType.DMA,
            pltpu.SemaphoreType.DMA,
        ],
    )(_onehot_sc)
    outt = fn(xt)
    return jnp.transpose(outt.reshape(_S, _D, _B), (2, 0, 1))


# speedup vs baseline: 5.0771x; 5.0771x over previous
import functools

import jax
import jax.numpy as jnp
from jax import lax
from jax.experimental import pallas as pl
from jax.experimental.pallas import tpu as pltpu
from jax.experimental.pallas import tpu_sc as plsc

_B, _S, _D = 1024, 50, 1000
_NR = _S * _D
_CH = 40
_CPS = _D // _CH
_NCH = _S * _CPS
_CPW = 40


def _onehot_sc(xt_hbm, out_hbm, xbuf, stage, sem0, sem1):
    cid = lax.axis_index("c")
    sid = lax.axis_index("s")
    wid = sid * 2 + cid
    c0 = (wid * _NCH) // 32

    s_first = c0 // _CPS
    s_last = (c0 + _CPW - 1) // _CPS
    a0 = pl.multiple_of((s_first // 8) * 8, 8)
    a1 = pl.multiple_of((s_last // 8) * 8, 8)
    pltpu.sync_copy(xt_hbm.at[pl.ds(a0, 8)], xbuf.at[pl.ds(0, 8)])
    pltpu.sync_copy(xt_hbm.at[pl.ds(a1, 8)], xbuf.at[pl.ds(8, 8)])

    sems = (sem0, sem1)

    def _fill(k, c):
        s = c // _CPS
        d0 = (c - s * _CPS) * _CH
        xrow = jnp.minimum(s - a0, s - a1 + 8)
        dsp = [jnp.full((16,), d0 + rr, jnp.int32) for rr in range(_CH)]

        def _grp(g, _):
            go = pl.multiple_of(g * 16, 16)
            xg = xbuf[xrow, pl.ds(go, 16)]
            for rr in range(_CH):
                stage[k, rr, pl.ds(go, 16)] = jnp.where(
                    xg == dsp[rr], 1.0, 0.0
                ).astype(jnp.float32)
            return 0

        lax.fori_loop(0, _B // 16, _grp, 0)

    for k in (0, 1):
        _fill(k, c0 + k)
        rb = pl.multiple_of((c0 + k) * _CH, 8)
        pltpu.async_copy(stage.at[k], out_hbm.at[pl.ds(rb, _CH)], sems[k])

    def _pair(cp, _):
        for k in (0, 1):
            c = c0 + 2 * cp + k
            rb = pl.multiple_of(c * _CH, 8)
            pltpu.make_async_copy(
                stage.at[k], out_hbm.at[pl.ds(0, _CH)], sems[k]
            ).wait()
            _fill(k, c)
            pltpu.async_copy(stage.at[k], out_hbm.at[pl.ds(rb, _CH)], sems[k])
        return 0

    lax.fori_loop(1, _CPW // 2, _pair, 0)

    pltpu.make_async_copy(stage.at[0], out_hbm.at[pl.ds(0, _CH)], sems[0]).wait()
    pltpu.make_async_copy(stage.at[1], out_hbm.at[pl.ds(0, _CH)], sems[1]).wait()


def kernel(x):
    xt = jnp.pad(x.astype(jnp.int32).T, ((0, 6), (0, 0)))
    mesh = plsc.VectorSubcoreMesh(
        core_axis_name="c", subcore_axis_name="s", num_cores=2
    )
    fn = functools.partial(
        pl.kernel,
        mesh=mesh,
        out_type=jax.ShapeDtypeStruct((_NR, _B), jnp.float32),
        scratch_types=[
            pltpu.VMEM((16, _B), jnp.int32),
            pltpu.VMEM((2, _CH, _B), jnp.float32),
            pltpu.SemaphoreType.DMA,
            pltpu.SemaphoreType.DMA,
        ],
    )(_onehot_sc)
    outt = fn(xt)
    return jnp.transpose(outt.reshape(_S, _D, _B), (2, 0, 1))
